# baseline (speedup 1.0000x reference)
import functools

import jax
import jax.numpy as jnp
from jax.experimental import pallas as pl
from jax.experimental.pallas import tpu as pltpu

_ROT0 = (13, 15, 26, 6)
_ROT1 = (17, 29, 16, 24)
_KS0 = 0
_KS1 = 42
_KS2 = _KS0 ^ _KS1 ^ 0x1BD11BDA
_TINY = 1.1754943508222875e-38


def _rotl(x, d):
    return (x << jnp.uint32(d)) | (x >> jnp.uint32(32 - d))


def _threefry_bits(p):
    ks = (jnp.uint32(_KS0), jnp.uint32(_KS1), jnp.uint32(_KS2))
    x0 = jnp.zeros_like(p) + ks[0]
    x1 = p + ks[1]
    for i in range(5):
        rots = _ROT0 if i % 2 == 0 else _ROT1
        for r in rots:
            x0 = x0 + x1
            x1 = _rotl(x1, r)
            x1 = x0 ^ x1
        x0 = x0 + ks[(i + 1) % 3]
        x1 = x1 + ks[(i + 2) % 3] + jnp.uint32(i + 1)
    return x0 ^ x1


def _gumbel_from_bits(bits):
    float_bits = (bits >> jnp.uint32(9)) | jnp.uint32(0x3F800000)
    floats = jax.lax.bitcast_convert_type(float_bits, jnp.float32) - jnp.float32(1.0)
    span = jnp.float32(1.0) - jnp.float32(_TINY)
    u = jnp.maximum(jnp.float32(_TINY), floats * span + jnp.float32(_TINY))
    return -jnp.log(-jnp.log(u))


def _sample_kernel(logits_ref, out_ref, *, vocab, block_rows, chunk, vpad):
    row0 = pl.program_id(0) * block_rows
    nchunks = pl.cdiv(vocab, chunk)
    rows = jax.lax.broadcasted_iota(jnp.int32, (block_rows, chunk), 0) + row0
    base_cols = jax.lax.broadcasted_iota(jnp.int32, (block_rows, chunk), 1)
    row_off = rows * vocab

    def body(c, carry):
        acc_val, acc_col = carry
        start = jnp.minimum(c * chunk, vpad - chunk)
        cols = base_cols + start
        p = (row_off + cols).astype(jnp.uint32)
        g = _gumbel_from_bits(_threefry_bits(p))
        vals = g + logits_ref[:, pl.ds(start, chunk)]
        vals = jnp.where(cols < vocab, vals, -jnp.inf)
        better = vals > acc_val
        acc_val = jnp.where(better, vals, acc_val)
        acc_col = jnp.where(better, cols, acc_col)
        return acc_val, acc_col

    init = (jnp.full((block_rows, chunk), -jnp.inf, jnp.float32),
            jnp.zeros((block_rows, chunk), jnp.int32))
    acc_val, acc_col = jax.lax.fori_loop(0, nchunks, body, init)

    m = jnp.max(acc_val, axis=1, keepdims=True)
    idx = jnp.min(jnp.where(acc_val == m, acc_col, jnp.int32(2**31 - 1)), axis=1)
    out_ref[...] = idx[:, None]


@jax.jit
def kernel(logits):
    b, vocab = logits.shape
    block_rows = 8
    chunk = 4096
    grid = (b // block_rows,)
    vpad = pl.cdiv(vocab, 128) * 128
    out = pl.pallas_call(
        functools.partial(_sample_kernel, vocab=vocab, block_rows=block_rows,
                          chunk=chunk, vpad=vpad),
        grid=grid,
        in_specs=[pl.BlockSpec((block_rows, vocab), lambda i: (i, 0))],
        out_specs=pl.BlockSpec((block_rows, 1), lambda i: (i, 0)),
        out_shape=jax.ShapeDtypeStruct((b, 1), jnp.int32),
        compiler_params=pltpu.CompilerParams(
            dimension_semantics=("parallel",),
        ),
    )(logits)
    return out[:, 0].astype(jnp.int64)


# speedup vs baseline: 1.1879x; 1.1879x over previous
import functools

import jax
import jax.numpy as jnp
from jax.experimental import pallas as pl
from jax.experimental.pallas import tpu as pltpu

_ROT0 = (13, 15, 26, 6)
_ROT1 = (17, 29, 16, 24)
_KS0 = 0
_KS1 = 42
_KS2 = _KS0 ^ _KS1 ^ 0x1BD11BDA
_TINY = 1.1754943508222875e-38


def _rotl(x, d):
    return (x << jnp.uint32(d)) | (x >> jnp.uint32(32 - d))


def _threefry_bits(p):
    ks = (jnp.uint32(_KS0), jnp.uint32(_KS1), jnp.uint32(_KS2))
    x0 = jnp.zeros_like(p) + ks[0]
    x1 = p + ks[1]
    for i in range(5):
        rots = _ROT0 if i % 2 == 0 else _ROT1
        for r in rots:
            x0 = x0 + x1
            x1 = _rotl(x1, r)
            x1 = x0 ^ x1
        x0 = x0 + ks[(i + 1) % 3]
        x1 = x1 + ks[(i + 2) % 3] + jnp.uint32(i + 1)
    return x0 ^ x1


def _gumbel_from_bits(bits):
    float_bits = (bits >> jnp.uint32(9)) | jnp.uint32(0x3F800000)
    floats = jax.lax.bitcast_convert_type(float_bits, jnp.float32) - jnp.float32(1.0)
    span = jnp.float32(1.0) - jnp.float32(_TINY)
    u = jnp.maximum(jnp.float32(_TINY), floats * span + jnp.float32(_TINY))
    return -jnp.log(-jnp.log(u))


def _sample_kernel(logits_ref, out_ref, *, vocab, block_rows, chunk, vpad):
    row0 = pl.program_id(0) * block_rows
    nchunks = pl.cdiv(vocab, chunk)
    rows = jax.lax.broadcasted_iota(jnp.int32, (block_rows, chunk), 0) + row0
    base_cols = jax.lax.broadcasted_iota(jnp.int32, (block_rows, chunk), 1)
    row_off = rows * vocab

    def body(c, carry):
        acc_val, acc_col = carry
        start = jnp.minimum(c * chunk, vpad - chunk)
        cols = base_cols + start
        p = (row_off + cols).astype(jnp.uint32)
        g = _gumbel_from_bits(_threefry_bits(p))
        vals = g + logits_ref[:, pl.ds(start, chunk)]
        vals = jnp.where(cols < vocab, vals, -jnp.inf)
        better = vals > acc_val
        acc_val = jnp.where(better, vals, acc_val)
        acc_col = jnp.where(better, cols, acc_col)
        return acc_val, acc_col

    init = (jnp.full((block_rows, chunk), -jnp.inf, jnp.float32),
            jnp.zeros((block_rows, chunk), jnp.int32))
    acc_val, acc_col = jax.lax.fori_loop(0, nchunks, body, init, unroll=4)

    m = jnp.max(acc_val, axis=1, keepdims=True)
    idx = jnp.min(jnp.where(acc_val == m, acc_col, jnp.int32(2**31 - 1)), axis=1)
    out_ref[...] = idx[:, None]


@jax.jit
def kernel(logits):
    b, vocab = logits.shape
    block_rows = 8
    chunk = 1024
    grid = (b // block_rows,)
    vpad = pl.cdiv(vocab, 128) * 128
    out = pl.pallas_call(
        functools.partial(_sample_kernel, vocab=vocab, block_rows=block_rows,
                          chunk=chunk, vpad=vpad),
        grid=grid,
        in_specs=[pl.BlockSpec((block_rows, vocab), lambda i: (i, 0))],
        out_specs=pl.BlockSpec((block_rows, 1), lambda i: (i, 0)),
        out_shape=jax.ShapeDtypeStruct((b, 1), jnp.int32),
        compiler_params=pltpu.CompilerParams(
            dimension_semantics=("parallel",),
        ),
    )(logits)
    return out[:, 0].astype(jnp.int64)
